# baseline (speedup 1.0000x reference)
import functools

import jax
import jax.numpy as jnp
from jax import lax
from jax.experimental import pallas as pl
from jax.experimental.pallas import tpu as pltpu
from jax.experimental.pallas import tpu_sc as plsc

SA_LEVELS = [
    (4096, 1024, 0.1, 3, (6, 32, 32, 64), 32, 8, 128, 128),
    (1024, 256, 0.2, 64, (67, 64, 64, 128), 8, 8, 128, 128),
    (256, 64, 0.4, 128, (131, 128, 128, 256), 2, 8, 64, 64),
    (64, 16, 0.8, 256, (259, 256, 256, 512), 1, 4, 16, 16),
]
NS = 32


def _rup(x, m):
    return (x + m - 1) // m * m


def _fps_body(x_ref, y_ref, z_ref, out_ref, dist_ref, *, S):
    B, N = x_ref.shape
    ids = lax.broadcasted_iota(jnp.int32, (B, N), 1)
    dist_ref[...] = jnp.full((B, N), 1e10, jnp.float32)

    def body(s, cur):
        x = x_ref[...]
        y = y_ref[...]
        z = z_ref[...]
        eq = ids == cur
        cx = jnp.sum(jnp.where(eq, x, 0.0), axis=1, keepdims=True)
        cy = jnp.sum(jnp.where(eq, y, 0.0), axis=1, keepdims=True)
        cz = jnp.sum(jnp.where(eq, z, 0.0), axis=1, keepdims=True)
        cent = jnp.concatenate([cx, cy, cz], axis=1)
        out_ref[pl.ds(s, 1), :, :] = cent[None]
        dx = x - cx
        dy = y - cy
        dz = z - cz
        d = (dx * dx + dy * dy) + dz * dz
        dmin = jnp.minimum(dist_ref[...], d)
        dist_ref[...] = dmin
        m = jnp.max(dmin, axis=1, keepdims=True)
        nxt = jnp.min(jnp.where(dmin == m, ids, N), axis=1, keepdims=True)
        return nxt

    lax.fori_loop(0, S, body, jnp.zeros((B, 1), jnp.int32))


def _fps(xyz, S):
    B, N, _ = xyz.shape
    xt = jnp.transpose(xyz, (2, 0, 1))
    out = pl.pallas_call(
        functools.partial(_fps_body, S=S),
        out_shape=jax.ShapeDtypeStruct((S, B, 3), jnp.float32),
        scratch_shapes=[pltpu.VMEM((B, N), jnp.float32)],
    )(xt[0], xt[1], xt[2])
    return jnp.transpose(out, (1, 0, 2))


def _shift_lanes(a, sh):
    z = jnp.zeros(a.shape[:-1] + (sh,), a.dtype)
    return jnp.concatenate([z, a[..., :-sh]], axis=-1)


def _shift_sub(a, sh):
    z = jnp.zeros((a.shape[0], sh, a.shape[2]), a.dtype)
    return jnp.concatenate([z, a[:, :-sh, :]], axis=1)


def _ballq_body(xt_ref, yt_ref, zt_ref, new_ref, out_ref, *, r2, W, U, Sblk):
    N = xt_ref.shape[2]
    K16 = U * 16
    x = xt_ref[0]
    y = yt_ref[0]
    z = zt_ref[0]
    new = new_ref[0]
    nx = new[:, 0:1]
    ny = new[:, 1:2]
    nz = new[:, 2:3]
    dx = nx - x
    dy = ny - y
    dz = nz - z
    sq = (dx * dx + dy * dy) + dz * dz
    mask = (sq < r2).astype(jnp.float32)

    m2 = mask.reshape(Sblk * W, K16)
    li = lax.broadcasted_iota(jnp.int32, (K16, 2 * U), 0)
    ui = lax.broadcasted_iota(jnp.int32, (K16, 2 * U), 1)
    ind = (li // 16) == (ui % U)
    kcnt = jnp.where(ind & (ui < U), 1.0, 0.0)
    kbit = jnp.where(ind & (ui >= U), (1 << (li % 16)).astype(jnp.float32), 0.0)
    tb = m2 @ (kcnt + kbit)
    tb3 = tb.reshape(Sblk, W, 2 * U)
    t2T = jnp.swapaxes(tb3[:, :, :U], 1, 2)
    bitsT = jnp.swapaxes(tb3[:, :, U:], 1, 2).astype(jnp.int32)

    t1 = jnp.sum(t2T, axis=1)
    H1 = t1
    sh = 1
    while sh < W:
        H1 = H1 + _shift_lanes(H1, sh)
        sh *= 2
    H1x = (H1 - t1).astype(jnp.int32)
    H1i = H1.astype(jnp.int32)
    count = H1i[:, W - 1:W]

    karr = lax.broadcasted_iota(jnp.int32, (Sblk, NS), 1)
    if W > 1:
        ws = jnp.sum((H1i[:, :, None] <= karr[:, None, :]).astype(jnp.int32),
                     axis=1)
        wsc = jnp.minimum(ws, W - 1)
        base1 = jnp.take_along_axis(H1x, wsc, axis=1)
        idxw = jnp.broadcast_to(wsc[:, None, :], (Sblk, U, NS))
        t2sel = jnp.take_along_axis(t2T.astype(jnp.int32), idxw, axis=2)
        bsel0 = jnp.take_along_axis(bitsT, idxw, axis=2)
    else:
        wsc = jnp.zeros((Sblk, NS), jnp.int32)
        base1 = jnp.zeros((Sblk, NS), jnp.int32)
        t2sel = jnp.broadcast_to(t2T.astype(jnp.int32), (Sblk, U, NS))
        bsel0 = jnp.broadcast_to(bitsT, (Sblk, U, NS))
    r1 = karr - base1

    if U > 1:
        H2 = t2sel
        sh = 1
        while sh < U:
            H2 = H2 + _shift_sub(H2, sh)
            sh *= 2
        H2x = H2 - t2sel
        us = jnp.sum((H2 <= r1[:, None, :]).astype(jnp.int32), axis=1)
        usc = jnp.minimum(us, U - 1)
        base2 = jnp.take_along_axis(H2x, usc[:, None, :], axis=1)[:, 0, :]
        bsel = jnp.take_along_axis(bsel0, usc[:, None, :], axis=1)[:, 0, :]
    else:
        usc = jnp.zeros((Sblk, NS), jnp.int32)
        base2 = jnp.zeros((Sblk, NS), jnp.int32)
        bsel = bsel0[:, 0, :]
    r2i = r1 - base2

    pos = jnp.zeros((Sblk, NS), jnp.int32)
    rem = r2i
    b = bsel
    for width in (8, 4, 2, 1):
        lowc = lax.population_count(b & ((1 << width) - 1))
        take = (rem >= lowc).astype(jnp.int32)
        pos = pos + take * width
        rem = rem - take * lowc
        b = lax.shift_right_logical(b, take * width)

    outk = wsc * (U * 16) + usc * 16 + pos
    outk = jnp.where(karr < count, outk, outk[:, 0:1])
    out_ref[0] = outk


def _ball_query(xyz, new_xyz, r, W, U, Sblk):
    B, N, _ = xyz.shape
    S = new_xyz.shape[1]
    xt = jnp.transpose(xyz, (2, 0, 1))[:, :, None, :]
    grid = (B, S // Sblk)
    return pl.pallas_call(
        functools.partial(_ballq_body, r2=r * r, W=W, U=U, Sblk=Sblk),
        grid=grid,
        compiler_params=pltpu.CompilerParams(
            dimension_semantics=("parallel", "arbitrary")),
        in_specs=[
            pl.BlockSpec((1, 1, N), lambda b, s: (b, 0, 0)),
            pl.BlockSpec((1, 1, N), lambda b, s: (b, 0, 0)),
            pl.BlockSpec((1, 1, N), lambda b, s: (b, 0, 0)),
            pl.BlockSpec((1, Sblk, 3), lambda b, s: (b, s, 0)),
        ],
        out_specs=pl.BlockSpec((1, Sblk, NS), lambda b, s: (b, s, 0)),
        out_shape=jax.ShapeDtypeStruct((B, S, NS), jnp.int32),
    )(xt[0], xt[1], xt[2], new_xyz)


def _sc_gather_call(table, idx_flat):
    V, D = table.shape
    Btot = idx_flat.shape[0]
    info = plsc.get_sparse_core_info()
    NW = info.num_cores * info.num_subcores
    b_per_w = Btot // NW
    CH = b_per_w
    while CH * D * 4 > 200_000:
        CH //= 2
    nch = b_per_w // CH
    mesh = plsc.VectorSubcoreMesh(core_axis_name="c", subcore_axis_name="s")

    @functools.partial(
        pl.kernel, mesh=mesh,
        compiler_params=pltpu.CompilerParams(use_tc_tiling_on_sc=False),
        out_type=jax.ShapeDtypeStruct((Btot, D), jnp.float32),
        scratch_types=[
            pltpu.VMEM((CH,), jnp.int32),
            pltpu.VMEM((CH,), jnp.int32),
            pltpu.VMEM((CH, D), jnp.float32),
            pltpu.VMEM((CH, D), jnp.float32),
            pltpu.SemaphoreType.DMA,
            pltpu.SemaphoreType.DMA,
        ],
    )
    def k(table_hbm, idx_hbm, out_hbm, idx_v0, idx_v1, rows_v0, rows_v1,
          sem0, sem1):
        wid = lax.axis_index("s") * info.num_cores + lax.axis_index("c")
        base = wid * b_per_w
        idx_bufs = (idx_v0, idx_v1)
        row_bufs = (rows_v0, rows_v1)
        sems = (sem0, sem1)
        copies = [None, None]
        pltpu.sync_copy(idx_hbm.at[pl.ds(base, CH)], idx_v0)
        copies[0] = pltpu.async_copy(table_hbm.at[idx_v0], rows_v0, sem0)
        for c in range(nch):
            nxt = (c + 1) % 2
            if c + 1 < nch:
                off = base + (c + 1) * CH
                pltpu.sync_copy(idx_hbm.at[pl.ds(off, CH)], idx_bufs[nxt])
                copies[nxt] = pltpu.async_copy(
                    table_hbm.at[idx_bufs[nxt]], row_bufs[nxt], sems[nxt])
            copies[c % 2].wait()
            pltpu.sync_copy(row_bufs[c % 2],
                            out_hbm.at[pl.ds(base + c * CH, CH)])

    return k(table, idx_flat)


def _sa_mlp_body(g_ref, new_ref, w1_ref, b1_ref, w2_ref, b2_ref, w3_ref,
                 b3_ref, out_ref, *, Sblk, Cout, Dpad):
    grows = g_ref[0].reshape(Sblk, NS, Dpad)
    new = new_ref[0]
    sub = jnp.concatenate(
        [new, jnp.zeros((Sblk, Dpad - 3), jnp.float32)], axis=1)
    xrows = (grows - sub[:, None, :]).reshape(Sblk * NS, Dpad)
    h = jnp.maximum(xrows @ w1_ref[...] + b1_ref[...], 0.0)
    h = jnp.maximum(h @ w2_ref[...] + b2_ref[...], 0.0)
    h = jnp.maximum(h @ w3_ref[...] + b3_ref[...], 0.0)
    hm = h.reshape(Sblk, NS, Cout)
    out_ref[0] = jnp.max(hm, axis=1)


def _sa_mlp(g, new_xyz, layers, Sblk):
    B, R, Dpad = g.shape
    S = R // NS
    ws = []
    for li, (Wm, bm) in enumerate(layers):
        cin, cout = Wm.shape
        rows_pad = Dpad if li == 0 else _rup(cin, 8)
        Wp = jnp.zeros((rows_pad, cout), jnp.float32).at[:cin].set(Wm)
        ws.append((Wp, bm.reshape(1, cout)))
    Cout = ws[-1][0].shape[1]
    grid = (B, S // Sblk)
    specs = [
        pl.BlockSpec((1, Sblk * NS, Dpad), lambda b, s: (b, s, 0)),
        pl.BlockSpec((1, Sblk, 3), lambda b, s: (b, s, 0)),
    ]
    args = [g, new_xyz]
    for (Wp, bp) in ws:
        specs.append(pl.BlockSpec(Wp.shape, lambda b, s: (0, 0)))
        specs.append(pl.BlockSpec(bp.shape, lambda b, s: (0, 0)))
        args.extend([Wp, bp])
    return pl.pallas_call(
        functools.partial(_sa_mlp_body, Sblk=Sblk, Cout=Cout, Dpad=Dpad),
        grid=grid,
        compiler_params=pltpu.CompilerParams(
            dimension_semantics=("parallel", "arbitrary")),
        in_specs=specs,
        out_specs=pl.BlockSpec((1, Sblk, Cout), lambda b, s: (b, s, 0)),
        out_shape=jax.ShapeDtypeStruct((B, S, Cout), jnp.float32),
    )(*args)


def _fp_body(x1_ref, q_ref, f2_ref, f1_ref, *rest, S2, C1, Cin_pad, nlayers,
             Nblk, Cout):
    wrefs = rest[:-1]
    out_ref = rest[-1]
    p = x1_ref[0]
    q = q_ref[0]
    pn = jnp.sum(p * p, axis=1, keepdims=True)
    qn = jnp.sum(q * q, axis=0, keepdims=True)
    pq = jax.lax.dot(p, q, precision=jax.lax.Precision.HIGHEST)
    d = (pn + qn) - 2.0 * pq
    lane = lax.broadcasted_iota(jnp.int32, (Nblk, S2), 1)

    def extract(dcur):
        dm = jnp.min(dcur, axis=1, keepdims=True)
        im = jnp.min(jnp.where(dcur == dm, lane, S2), axis=1, keepdims=True)
        eq = lane == im
        dnext = jnp.where(eq, 1e30, dcur)
        return dm, eq, dnext

    d1, e1, dc = extract(d)
    d2, e2, dc = extract(dc)
    d3, e3, _ = extract(dc)
    r1 = 1.0 / (d1 + 1e-8)
    r2 = 1.0 / (d2 + 1e-8)
    r3 = 1.0 / (d3 + 1e-8)
    rs = r1 + r2 + r3
    w1 = r1 / rs
    w2 = r2 / rs
    w3 = r3 / rs
    Wm = (jnp.where(e1, w1, 0.0) + jnp.where(e2, w2, 0.0)
          + jnp.where(e3, w3, 0.0))
    interp = jax.lax.dot(Wm, f2_ref[0],
                         precision=jax.lax.Precision.HIGHEST)
    xin = jnp.concatenate([interp, f1_ref[0]], axis=1)
    pad = Cin_pad - xin.shape[1]
    if pad:
        xin = jnp.concatenate(
            [xin, jnp.zeros((Nblk, pad), jnp.float32)], axis=1)
    h = xin
    for i in range(nlayers):
        h = jnp.maximum(h @ wrefs[2 * i][...] + wrefs[2 * i + 1][...], 0.0)
    out_ref[0] = h


def _fp(xyz1, xyz2, feats1, feats2, layers, Nblk):
    B, N1, _ = xyz1.shape
    S2 = xyz2.shape[1]
    C1 = feats1.shape[2]
    C2 = feats2.shape[2]
    q = jnp.transpose(xyz2, (0, 2, 1))
    Cin = C1 + C2
    Cin_pad = _rup(Cin, 8)
    ws = []
    for (Wm, bm) in layers:
        cin, cout = Wm.shape
        Wp = jnp.zeros((_rup(cin, 8), cout), jnp.float32).at[:cin].set(Wm)
        ws.append((Wp, bm.reshape(1, cout)))
    nlayers = len(ws)
    Cout = ws[-1][0].shape[1]
    grid = (B, N1 // Nblk)
    specs = [
        pl.BlockSpec((1, Nblk, 3), lambda b, s: (b, s, 0)),
        pl.BlockSpec((1, 3, S2), lambda b, s: (b, 0, 0)),
        pl.BlockSpec((1, S2, C2), lambda b, s: (b, 0, 0)),
        pl.BlockSpec((1, Nblk, C1), lambda b, s: (b, s, 0)),
    ]
    args = [xyz1, q, feats2, feats1]
    for (Wp, bp) in ws:
        specs.append(pl.BlockSpec(Wp.shape, lambda b, s: (0, 0)))
        specs.append(pl.BlockSpec(bp.shape, lambda b, s: (0, 0)))
        args.extend([Wp, bp])
    return pl.pallas_call(
        functools.partial(_fp_body, S2=S2, C1=C1, Cin_pad=Cin_pad,
                          nlayers=nlayers, Nblk=Nblk, Cout=Cout),
        grid=grid,
        compiler_params=pltpu.CompilerParams(
            dimension_semantics=("parallel", "arbitrary")),
        in_specs=specs,
        out_specs=pl.BlockSpec((1, Nblk, Cout), lambda b, s: (b, s, 0)),
        out_shape=jax.ShapeDtypeStruct((B, N1, Cout), jnp.float32),
    )(*args)


def kernel(pointcloud, params):
    B = pointcloud.shape[0]
    xyz = pointcloud[..., 0:3]
    feats = pointcloud[..., 3:]

    l_xyz = [xyz]
    l_feats = [feats]
    for lvl, (N, S, r, Cf, dims, W, U, Sblk_bq, Sblk_mlp) in enumerate(
            SA_LEVELS):
        cur_xyz = l_xyz[lvl]
        cur_f = l_feats[lvl]
        new_xyz = _fps(cur_xyz, S)
        idx = _ball_query(cur_xyz, new_xyz, r, W, U, Sblk_bq)
        Dpad = _rup(3 + Cf, 16)
        tab = jnp.concatenate([cur_xyz, cur_f], axis=-1)
        tab = jnp.pad(tab, ((0, 0), (0, 0), (0, Dpad - (3 + Cf))))
        tab = tab.reshape(B * N, Dpad)
        glob = (idx + jnp.arange(B, dtype=jnp.int32)[:, None, None] * N)
        rows = _sc_gather_call(tab, glob.reshape(-1))
        g = rows.reshape(B, S * NS, Dpad)
        nf = _sa_mlp(g, new_xyz, params["sa"][lvl], Sblk_mlp)
        l_xyz.append(new_xyz)
        l_feats.append(nf)

    fp_nblk = [512, 256, 256, 64]
    for i in range(-1, -5, -1):
        l_feats[i - 1] = _fp(
            l_xyz[i - 1], l_xyz[i], l_feats[i - 1], l_feats[i],
            params["fp"][i], fp_nblk[i])
    return l_feats[0]


# speedup vs baseline: 1.1256x; 1.1256x over previous
import functools

import jax
import jax.numpy as jnp
from jax import lax
from jax.experimental import pallas as pl
from jax.experimental.pallas import tpu as pltpu
from jax.experimental.pallas import tpu_sc as plsc

SA_LEVELS = [
    (4096, 1024, 0.1, 3, (6, 32, 32, 64), 32, 8, 128, 128),
    (1024, 256, 0.2, 64, (67, 64, 64, 128), 8, 8, 128, 128),
    (256, 64, 0.4, 128, (131, 128, 128, 256), 2, 8, 64, 64),
    (64, 16, 0.8, 256, (259, 256, 256, 512), 1, 4, 16, 16),
]
NS = 32


def _rup(x, m):
    return (x + m - 1) // m * m


def _fps_body(x_ref, y_ref, z_ref, out_ref, dist_ref, *, S):
    B, N = x_ref.shape
    ids = lax.broadcasted_iota(jnp.int32, (B, N), 1)
    dist_ref[...] = jnp.full((B, N), 1e10, jnp.float32)

    def body(s, cur):
        x = x_ref[...]
        y = y_ref[...]
        z = z_ref[...]
        eq = ids == cur
        cx = jnp.sum(jnp.where(eq, x, 0.0), axis=1, keepdims=True)
        cy = jnp.sum(jnp.where(eq, y, 0.0), axis=1, keepdims=True)
        cz = jnp.sum(jnp.where(eq, z, 0.0), axis=1, keepdims=True)
        cent = jnp.concatenate([cx, cy, cz], axis=1)
        out_ref[pl.ds(s, 1), :, :] = cent[None]
        dx = x - cx
        dy = y - cy
        dz = z - cz
        d = (dx * dx + dy * dy) + dz * dz
        dmin = jnp.minimum(dist_ref[...], d)
        dist_ref[...] = dmin
        m = jnp.max(dmin, axis=1, keepdims=True)
        nxt = jnp.min(jnp.where(dmin == m, ids, N), axis=1, keepdims=True)
        return nxt

    lax.fori_loop(0, S, body, jnp.zeros((B, 1), jnp.int32))


def _fps(xyz, S):
    B, N, _ = xyz.shape
    xt = jnp.transpose(xyz, (2, 0, 1))
    out = pl.pallas_call(
        functools.partial(_fps_body, S=S),
        out_shape=jax.ShapeDtypeStruct((S, B, 3), jnp.float32),
        scratch_shapes=[pltpu.VMEM((B, N), jnp.float32)],
    )(xt[0], xt[1], xt[2])
    return jnp.transpose(out, (1, 0, 2))


def _shift_lanes(a, sh):
    z = jnp.zeros(a.shape[:-1] + (sh,), a.dtype)
    return jnp.concatenate([z, a[..., :-sh]], axis=-1)


def _shift_sub(a, sh):
    z = jnp.zeros((a.shape[0], sh, a.shape[2]), a.dtype)
    return jnp.concatenate([z, a[:, :-sh, :]], axis=1)


def _ballq_body(xt_ref, yt_ref, zt_ref, new_ref, out_ref, *, r2, W, U, Sblk):
    N = xt_ref.shape[2]
    K16 = U * 16
    x = xt_ref[0]
    y = yt_ref[0]
    z = zt_ref[0]
    new = new_ref[0]
    nx = new[:, 0:1]
    ny = new[:, 1:2]
    nz = new[:, 2:3]
    dx = nx - x
    dy = ny - y
    dz = nz - z
    sq = (dx * dx + dy * dy) + dz * dz
    mask = (sq < r2).astype(jnp.float32)

    m2 = mask.reshape(Sblk * W, K16)
    li = lax.broadcasted_iota(jnp.int32, (K16, 2 * U), 0)
    ui = lax.broadcasted_iota(jnp.int32, (K16, 2 * U), 1)
    ind = (li // 16) == (ui % U)
    kcnt = jnp.where(ind & (ui < U), 1.0, 0.0)
    kbit = jnp.where(ind & (ui >= U), (1 << (li % 16)).astype(jnp.float32), 0.0)
    tb = m2 @ (kcnt + kbit)
    tb3 = tb.reshape(Sblk, W, 2 * U)
    t2T = jnp.swapaxes(tb3[:, :, :U], 1, 2)
    bitsT = jnp.swapaxes(tb3[:, :, U:], 1, 2).astype(jnp.int32)

    t1 = jnp.sum(t2T, axis=1)
    H1 = t1
    sh = 1
    while sh < W:
        H1 = H1 + _shift_lanes(H1, sh)
        sh *= 2
    H1x = (H1 - t1).astype(jnp.int32)
    H1i = H1.astype(jnp.int32)
    count = H1i[:, W - 1:W]

    karr = lax.broadcasted_iota(jnp.int32, (Sblk, NS), 1)
    if W > 1:
        ws = jnp.sum((H1i[:, :, None] <= karr[:, None, :]).astype(jnp.int32),
                     axis=1)
        wsc = jnp.minimum(ws, W - 1)
        base1 = jnp.take_along_axis(H1x, wsc, axis=1)
        idxw = jnp.broadcast_to(wsc[:, None, :], (Sblk, U, NS))
        t2sel = jnp.take_along_axis(t2T.astype(jnp.int32), idxw, axis=2)
        bsel0 = jnp.take_along_axis(bitsT, idxw, axis=2)
    else:
        wsc = jnp.zeros((Sblk, NS), jnp.int32)
        base1 = jnp.zeros((Sblk, NS), jnp.int32)
        t2sel = jnp.broadcast_to(t2T.astype(jnp.int32), (Sblk, U, NS))
        bsel0 = jnp.broadcast_to(bitsT, (Sblk, U, NS))
    r1 = karr - base1

    if U > 1:
        H2 = t2sel
        sh = 1
        while sh < U:
            H2 = H2 + _shift_sub(H2, sh)
            sh *= 2
        H2x = H2 - t2sel
        us = jnp.sum((H2 <= r1[:, None, :]).astype(jnp.int32), axis=1)
        usc = jnp.minimum(us, U - 1)
        base2 = jnp.take_along_axis(H2x, usc[:, None, :], axis=1)[:, 0, :]
        bsel = jnp.take_along_axis(bsel0, usc[:, None, :], axis=1)[:, 0, :]
    else:
        usc = jnp.zeros((Sblk, NS), jnp.int32)
        base2 = jnp.zeros((Sblk, NS), jnp.int32)
        bsel = bsel0[:, 0, :]
    r2i = r1 - base2

    pos = jnp.zeros((Sblk, NS), jnp.int32)
    rem = r2i
    b = bsel
    for width in (8, 4, 2, 1):
        lowc = lax.population_count(b & ((1 << width) - 1))
        take = (rem >= lowc).astype(jnp.int32)
        pos = pos + take * width
        rem = rem - take * lowc
        b = lax.shift_right_logical(b, take * width)

    outk = wsc * (U * 16) + usc * 16 + pos
    outk = jnp.where(karr < count, outk, outk[:, 0:1])
    out_ref[0] = outk


def _ball_query(xyz, new_xyz, r, W, U, Sblk):
    B, N, _ = xyz.shape
    S = new_xyz.shape[1]
    xt = jnp.transpose(xyz, (2, 0, 1))[:, :, None, :]
    grid = (B, S // Sblk)
    return pl.pallas_call(
        functools.partial(_ballq_body, r2=r * r, W=W, U=U, Sblk=Sblk),
        grid=grid,
        compiler_params=pltpu.CompilerParams(
            dimension_semantics=("parallel", "arbitrary")),
        in_specs=[
            pl.BlockSpec((1, 1, N), lambda b, s: (b, 0, 0)),
            pl.BlockSpec((1, 1, N), lambda b, s: (b, 0, 0)),
            pl.BlockSpec((1, 1, N), lambda b, s: (b, 0, 0)),
            pl.BlockSpec((1, Sblk, 3), lambda b, s: (b, s, 0)),
        ],
        out_specs=pl.BlockSpec((1, Sblk, NS), lambda b, s: (b, s, 0)),
        out_shape=jax.ShapeDtypeStruct((B, S, NS), jnp.int32),
    )(xt[0], xt[1], xt[2], new_xyz)


def _sc_gather_call(table, idx_flat):
    V, D = table.shape
    Btot = idx_flat.shape[0]
    info = plsc.get_sparse_core_info()
    NW = info.num_cores * info.num_subcores
    b_per_w = Btot // NW
    CH = b_per_w
    while CH * D * 4 > 200_000:
        CH //= 2
    nch = b_per_w // CH
    mesh = plsc.VectorSubcoreMesh(core_axis_name="c", subcore_axis_name="s")

    @functools.partial(
        pl.kernel, mesh=mesh,
        compiler_params=pltpu.CompilerParams(use_tc_tiling_on_sc=False),
        out_type=jax.ShapeDtypeStruct((Btot, D), jnp.float32),
        scratch_types=[
            pltpu.VMEM((CH,), jnp.int32),
            pltpu.VMEM((CH,), jnp.int32),
            pltpu.VMEM((CH, D), jnp.float32),
            pltpu.VMEM((CH, D), jnp.float32),
            pltpu.SemaphoreType.DMA,
            pltpu.SemaphoreType.DMA,
        ],
    )
    def k(table_hbm, idx_hbm, out_hbm, idx_v0, idx_v1, rows_v0, rows_v1,
          sem0, sem1):
        wid = lax.axis_index("s") * info.num_cores + lax.axis_index("c")
        base = wid * b_per_w
        idx_bufs = (idx_v0, idx_v1)
        row_bufs = (rows_v0, rows_v1)
        sems = (sem0, sem1)
        copies = [None, None]
        pltpu.sync_copy(idx_hbm.at[pl.ds(base, CH)], idx_v0)
        copies[0] = pltpu.async_copy(table_hbm.at[idx_v0], rows_v0, sem0)
        for c in range(nch):
            nxt = (c + 1) % 2
            if c + 1 < nch:
                off = base + (c + 1) * CH
                pltpu.sync_copy(idx_hbm.at[pl.ds(off, CH)], idx_bufs[nxt])
                copies[nxt] = pltpu.async_copy(
                    table_hbm.at[idx_bufs[nxt]], row_bufs[nxt], sems[nxt])
            copies[c % 2].wait()
            pltpu.sync_copy(row_bufs[c % 2],
                            out_hbm.at[pl.ds(base + c * CH, CH)])

    return k(table, idx_flat)


def _sa_mlp_body(g_ref, new_ref, w1_ref, b1_ref, w2_ref, b2_ref, w3_ref,
                 b3_ref, out_ref, *, Sblk, Cout, Dpad):
    grows = g_ref[0].reshape(Sblk, NS, Dpad)
    new = new_ref[0]
    sub = jnp.concatenate(
        [new, jnp.zeros((Sblk, Dpad - 3), jnp.float32)], axis=1)
    xrows = (grows - sub[:, None, :]).reshape(Sblk * NS, Dpad)
    h = jnp.maximum(xrows @ w1_ref[...] + b1_ref[...], 0.0)
    h = jnp.maximum(h @ w2_ref[...] + b2_ref[...], 0.0)
    h = jnp.maximum(h @ w3_ref[...] + b3_ref[...], 0.0)
    hm = h.reshape(Sblk, NS, Cout)
    out_ref[0] = jnp.max(hm, axis=1)


def _sa_mlp(g, new_xyz, layers, Sblk):
    B, R, Dpad = g.shape
    S = R // NS
    ws = []
    for li, (Wm, bm) in enumerate(layers):
        cin, cout = Wm.shape
        rows_pad = Dpad if li == 0 else _rup(cin, 8)
        Wp = jnp.zeros((rows_pad, cout), jnp.float32).at[:cin].set(Wm)
        ws.append((Wp, bm.reshape(1, cout)))
    Cout = ws[-1][0].shape[1]
    grid = (B, S // Sblk)
    specs = [
        pl.BlockSpec((1, Sblk * NS, Dpad), lambda b, s: (b, s, 0)),
        pl.BlockSpec((1, Sblk, 3), lambda b, s: (b, s, 0)),
    ]
    args = [g, new_xyz]
    for (Wp, bp) in ws:
        specs.append(pl.BlockSpec(Wp.shape, lambda b, s: (0, 0)))
        specs.append(pl.BlockSpec(bp.shape, lambda b, s: (0, 0)))
        args.extend([Wp, bp])
    return pl.pallas_call(
        functools.partial(_sa_mlp_body, Sblk=Sblk, Cout=Cout, Dpad=Dpad),
        grid=grid,
        compiler_params=pltpu.CompilerParams(
            dimension_semantics=("parallel", "arbitrary")),
        in_specs=specs,
        out_specs=pl.BlockSpec((1, Sblk, Cout), lambda b, s: (b, s, 0)),
        out_shape=jax.ShapeDtypeStruct((B, S, Cout), jnp.float32),
    )(*args)


def _fp_body(x1_ref, q_ref, f2_ref, f1_ref, *rest, S2, C1, Cin_pad, nlayers,
             Nblk, Cout):
    wrefs = rest[:-1]
    out_ref = rest[-1]
    p = x1_ref[0]
    q = q_ref[0]
    dx = p[:, 0:1] - q[0:1, :]
    dy = p[:, 1:2] - q[1:2, :]
    dz = p[:, 2:3] - q[2:3, :]
    d = (dx * dx + dy * dy) + dz * dz
    lane = lax.broadcasted_iota(jnp.int32, (Nblk, S2), 1)

    def extract(dcur):
        dm = jnp.min(dcur, axis=1, keepdims=True)
        im = jnp.min(jnp.where(dcur == dm, lane, S2), axis=1, keepdims=True)
        eq = lane == im
        dnext = jnp.where(eq, 1e30, dcur)
        return dm, eq, dnext

    d1, e1, dc = extract(d)
    d2, e2, dc = extract(dc)
    d3, e3, _ = extract(dc)
    r1 = 1.0 / (d1 + 1e-8)
    r2 = 1.0 / (d2 + 1e-8)
    r3 = 1.0 / (d3 + 1e-8)
    rs = r1 + r2 + r3
    w1 = r1 / rs
    w2 = r2 / rs
    w3 = r3 / rs
    Wm = (jnp.where(e1, w1, 0.0) + jnp.where(e2, w2, 0.0)
          + jnp.where(e3, w3, 0.0))
    interp = jax.lax.dot(Wm, f2_ref[0],
                         precision=jax.lax.Precision.HIGHEST)
    xin = jnp.concatenate([interp, f1_ref[0]], axis=1)
    pad = Cin_pad - xin.shape[1]
    if pad:
        xin = jnp.concatenate(
            [xin, jnp.zeros((Nblk, pad), jnp.float32)], axis=1)
    h = xin
    for i in range(nlayers):
        h = jnp.maximum(h @ wrefs[2 * i][...] + wrefs[2 * i + 1][...], 0.0)
    out_ref[0] = h


def _fp(xyz1, xyz2, feats1, feats2, layers, Nblk):
    B, N1, _ = xyz1.shape
    S2 = xyz2.shape[1]
    C1 = feats1.shape[2]
    C2 = feats2.shape[2]
    q = jnp.transpose(xyz2, (0, 2, 1))
    Cin = C1 + C2
    Cin_pad = _rup(Cin, 8)
    ws = []
    for (Wm, bm) in layers:
        cin, cout = Wm.shape
        Wp = jnp.zeros((_rup(cin, 8), cout), jnp.float32).at[:cin].set(Wm)
        ws.append((Wp, bm.reshape(1, cout)))
    nlayers = len(ws)
    Cout = ws[-1][0].shape[1]
    grid = (B, N1 // Nblk)
    specs = [
        pl.BlockSpec((1, Nblk, 3), lambda b, s: (b, s, 0)),
        pl.BlockSpec((1, 3, S2), lambda b, s: (b, 0, 0)),
        pl.BlockSpec((1, S2, C2), lambda b, s: (b, 0, 0)),
        pl.BlockSpec((1, Nblk, C1), lambda b, s: (b, s, 0)),
    ]
    args = [xyz1, q, feats2, feats1]
    for (Wp, bp) in ws:
        specs.append(pl.BlockSpec(Wp.shape, lambda b, s: (0, 0)))
        specs.append(pl.BlockSpec(bp.shape, lambda b, s: (0, 0)))
        args.extend([Wp, bp])
    return pl.pallas_call(
        functools.partial(_fp_body, S2=S2, C1=C1, Cin_pad=Cin_pad,
                          nlayers=nlayers, Nblk=Nblk, Cout=Cout),
        grid=grid,
        compiler_params=pltpu.CompilerParams(
            dimension_semantics=("parallel", "arbitrary")),
        in_specs=specs,
        out_specs=pl.BlockSpec((1, Nblk, Cout), lambda b, s: (b, s, 0)),
        out_shape=jax.ShapeDtypeStruct((B, N1, Cout), jnp.float32),
    )(*args)


def kernel(pointcloud, params):
    B = pointcloud.shape[0]
    xyz = pointcloud[..., 0:3]
    feats = pointcloud[..., 3:]

    l_xyz = [xyz]
    l_feats = [feats]
    for lvl, (N, S, r, Cf, dims, W, U, Sblk_bq, Sblk_mlp) in enumerate(
            SA_LEVELS):
        cur_xyz = l_xyz[lvl]
        cur_f = l_feats[lvl]
        new_xyz = _fps(cur_xyz, S)
        idx = _ball_query(cur_xyz, new_xyz, r, W, U, Sblk_bq)
        Dpad = _rup(3 + Cf, 16)
        tab = jnp.concatenate([cur_xyz, cur_f], axis=-1)
        tab = jnp.pad(tab, ((0, 0), (0, 0), (0, Dpad - (3 + Cf))))
        tab = tab.reshape(B * N, Dpad)
        glob = (idx + jnp.arange(B, dtype=jnp.int32)[:, None, None] * N)
        rows = _sc_gather_call(tab, glob.reshape(-1))
        g = rows.reshape(B, S * NS, Dpad)
        nf = _sa_mlp(g, new_xyz, params["sa"][lvl], Sblk_mlp)
        l_xyz.append(new_xyz)
        l_feats.append(nf)

    fp_nblk = [512, 256, 256, 64]
    for i in range(-1, -5, -1):
        l_feats[i - 1] = _fp(
            l_xyz[i - 1], l_xyz[i], l_feats[i - 1], l_feats[i],
            params["fp"][i], fp_nblk[i])
    return l_feats[0]


# speedup vs baseline: 1.1558x; 1.0268x over previous
import functools

import jax
import jax.numpy as jnp
from jax import lax
from jax.experimental import pallas as pl
from jax.experimental.pallas import tpu as pltpu
from jax.experimental.pallas import tpu_sc as plsc

SA_LEVELS = [
    (4096, 1024, 0.1, 3, (6, 32, 32, 64), 32, 8, 256, 256),
    (1024, 256, 0.2, 64, (67, 64, 64, 128), 8, 8, 128, 128),
    (256, 64, 0.4, 128, (131, 128, 128, 256), 2, 8, 64, 64),
    (64, 16, 0.8, 256, (259, 256, 256, 512), 1, 4, 16, 16),
]
NS = 32


def _rup(x, m):
    return (x + m - 1) // m * m


def _fps_body(x_ref, y_ref, z_ref, out_ref, dist_ref, *, S):
    B, N = x_ref.shape
    ids = lax.broadcasted_iota(jnp.int32, (B, N), 1)
    dist_ref[...] = jnp.full((B, N), 1e10, jnp.float32)

    def body(s, cur):
        x = x_ref[...]
        y = y_ref[...]
        z = z_ref[...]
        eq = ids == cur
        cx = jnp.sum(jnp.where(eq, x, 0.0), axis=1, keepdims=True)
        cy = jnp.sum(jnp.where(eq, y, 0.0), axis=1, keepdims=True)
        cz = jnp.sum(jnp.where(eq, z, 0.0), axis=1, keepdims=True)
        cent = jnp.concatenate([cx, cy, cz], axis=1)
        out_ref[pl.ds(s, 1), :, :] = cent[None]
        dx = x - cx
        dy = y - cy
        dz = z - cz
        d = (dx * dx + dy * dy) + dz * dz
        dmin = jnp.minimum(dist_ref[...], d)
        dist_ref[...] = dmin
        m = jnp.max(dmin, axis=1, keepdims=True)
        nxt = jnp.min(jnp.where(dmin == m, ids, N), axis=1, keepdims=True)
        return nxt

    lax.fori_loop(0, S, body, jnp.zeros((B, 1), jnp.int32))


def _fps(xyz, S):
    B, N, _ = xyz.shape
    xt = jnp.transpose(xyz, (2, 0, 1))
    out = pl.pallas_call(
        functools.partial(_fps_body, S=S),
        out_shape=jax.ShapeDtypeStruct((S, B, 3), jnp.float32),
        scratch_shapes=[pltpu.VMEM((B, N), jnp.float32)],
    )(xt[0], xt[1], xt[2])
    return jnp.transpose(out, (1, 0, 2))


def _shift_lanes(a, sh):
    z = jnp.zeros(a.shape[:-1] + (sh,), a.dtype)
    return jnp.concatenate([z, a[..., :-sh]], axis=-1)


def _shift_sub(a, sh):
    z = jnp.zeros((a.shape[0], sh, a.shape[2]), a.dtype)
    return jnp.concatenate([z, a[:, :-sh, :]], axis=1)


def _ballq_body(xt_ref, yt_ref, zt_ref, new_ref, out_ref, *, r2, W, U, Sblk):
    N = xt_ref.shape[2]
    K16 = U * 16
    x = xt_ref[0]
    y = yt_ref[0]
    z = zt_ref[0]
    new = new_ref[0]
    nx = new[:, 0:1]
    ny = new[:, 1:2]
    nz = new[:, 2:3]
    dx = nx - x
    dy = ny - y
    dz = nz - z
    sq = (dx * dx + dy * dy) + dz * dz
    mask = (sq < r2).astype(jnp.float32)

    m2 = mask.reshape(Sblk * W, K16)
    li = lax.broadcasted_iota(jnp.int32, (K16, 2 * U), 0)
    ui = lax.broadcasted_iota(jnp.int32, (K16, 2 * U), 1)
    ind = (li // 16) == (ui % U)
    kcnt = jnp.where(ind & (ui < U), 1.0, 0.0)
    kbit = jnp.where(ind & (ui >= U), (1 << (li % 16)).astype(jnp.float32), 0.0)
    tb = m2 @ (kcnt + kbit)
    tb3 = tb.reshape(Sblk, W, 2 * U)
    t2T = jnp.swapaxes(tb3[:, :, :U], 1, 2)
    bitsT = jnp.swapaxes(tb3[:, :, U:], 1, 2).astype(jnp.int32)

    t1 = jnp.sum(t2T, axis=1)
    H1 = t1
    sh = 1
    while sh < W:
        H1 = H1 + _shift_lanes(H1, sh)
        sh *= 2
    H1x = (H1 - t1).astype(jnp.int32)
    H1i = H1.astype(jnp.int32)
    count = H1i[:, W - 1:W]

    karr = lax.broadcasted_iota(jnp.int32, (Sblk, NS), 1)
    if W > 1:
        ws = jnp.sum((H1i[:, :, None] <= karr[:, None, :]).astype(jnp.int32),
                     axis=1)
        wsc = jnp.minimum(ws, W - 1)
        base1 = jnp.take_along_axis(H1x, wsc, axis=1)
        idxw = jnp.broadcast_to(wsc[:, None, :], (Sblk, U, NS))
        t2sel = jnp.take_along_axis(t2T.astype(jnp.int32), idxw, axis=2)
        bsel0 = jnp.take_along_axis(bitsT, idxw, axis=2)
    else:
        wsc = jnp.zeros((Sblk, NS), jnp.int32)
        base1 = jnp.zeros((Sblk, NS), jnp.int32)
        t2sel = jnp.broadcast_to(t2T.astype(jnp.int32), (Sblk, U, NS))
        bsel0 = jnp.broadcast_to(bitsT, (Sblk, U, NS))
    r1 = karr - base1

    if U > 1:
        H2 = t2sel
        sh = 1
        while sh < U:
            H2 = H2 + _shift_sub(H2, sh)
            sh *= 2
        H2x = H2 - t2sel
        us = jnp.sum((H2 <= r1[:, None, :]).astype(jnp.int32), axis=1)
        usc = jnp.minimum(us, U - 1)
        base2 = jnp.take_along_axis(H2x, usc[:, None, :], axis=1)[:, 0, :]
        bsel = jnp.take_along_axis(bsel0, usc[:, None, :], axis=1)[:, 0, :]
    else:
        usc = jnp.zeros((Sblk, NS), jnp.int32)
        base2 = jnp.zeros((Sblk, NS), jnp.int32)
        bsel = bsel0[:, 0, :]
    r2i = r1 - base2

    pos = jnp.zeros((Sblk, NS), jnp.int32)
    rem = r2i
    b = bsel
    for width in (8, 4, 2, 1):
        lowc = lax.population_count(b & ((1 << width) - 1))
        take = (rem >= lowc).astype(jnp.int32)
        pos = pos + take * width
        rem = rem - take * lowc
        b = lax.shift_right_logical(b, take * width)

    outk = wsc * (U * 16) + usc * 16 + pos
    outk = jnp.where(karr < count, outk, outk[:, 0:1])
    out_ref[0] = outk


def _ball_query(xyz, new_xyz, r, W, U, Sblk):
    B, N, _ = xyz.shape
    S = new_xyz.shape[1]
    xt = jnp.transpose(xyz, (2, 0, 1))[:, :, None, :]
    grid = (B, S // Sblk)
    return pl.pallas_call(
        functools.partial(_ballq_body, r2=r * r, W=W, U=U, Sblk=Sblk),
        grid=grid,
        compiler_params=pltpu.CompilerParams(
            dimension_semantics=("parallel", "arbitrary")),
        in_specs=[
            pl.BlockSpec((1, 1, N), lambda b, s: (b, 0, 0)),
            pl.BlockSpec((1, 1, N), lambda b, s: (b, 0, 0)),
            pl.BlockSpec((1, 1, N), lambda b, s: (b, 0, 0)),
            pl.BlockSpec((1, Sblk, 3), lambda b, s: (b, s, 0)),
        ],
        out_specs=pl.BlockSpec((1, Sblk, NS), lambda b, s: (b, s, 0)),
        out_shape=jax.ShapeDtypeStruct((B, S, NS), jnp.int32),
    )(xt[0], xt[1], xt[2], new_xyz)


def _sc_gather_call(table, idx_flat):
    V, D = table.shape
    Btot = idx_flat.shape[0]
    info = plsc.get_sparse_core_info()
    NW = info.num_cores * info.num_subcores
    b_per_w = Btot // NW
    CH = b_per_w
    while CH * D * 4 > 200_000:
        CH //= 2
    nch = b_per_w // CH
    mesh = plsc.VectorSubcoreMesh(core_axis_name="c", subcore_axis_name="s")

    @functools.partial(
        pl.kernel, mesh=mesh,
        compiler_params=pltpu.CompilerParams(use_tc_tiling_on_sc=False),
        out_type=jax.ShapeDtypeStruct((Btot, D), jnp.float32),
        scratch_types=[
            pltpu.VMEM((CH,), jnp.int32),
            pltpu.VMEM((CH,), jnp.int32),
            pltpu.VMEM((CH, D), jnp.float32),
            pltpu.VMEM((CH, D), jnp.float32),
            pltpu.SemaphoreType.DMA,
            pltpu.SemaphoreType.DMA,
        ],
    )
    def k(table_hbm, idx_hbm, out_hbm, idx_v0, idx_v1, rows_v0, rows_v1,
          sem0, sem1):
        wid = lax.axis_index("s") * info.num_cores + lax.axis_index("c")
        base = wid * b_per_w
        idx_bufs = (idx_v0, idx_v1)
        row_bufs = (rows_v0, rows_v1)
        sems = (sem0, sem1)
        copies = [None, None]
        pltpu.sync_copy(idx_hbm.at[pl.ds(base, CH)], idx_v0)
        copies[0] = pltpu.async_copy(table_hbm.at[idx_v0], rows_v0, sem0)
        for c in range(nch):
            nxt = (c + 1) % 2
            if c + 1 < nch:
                off = base + (c + 1) * CH
                pltpu.sync_copy(idx_hbm.at[pl.ds(off, CH)], idx_bufs[nxt])
                copies[nxt] = pltpu.async_copy(
                    table_hbm.at[idx_bufs[nxt]], row_bufs[nxt], sems[nxt])
            copies[c % 2].wait()
            pltpu.sync_copy(row_bufs[c % 2],
                            out_hbm.at[pl.ds(base + c * CH, CH)])

    return k(table, idx_flat)


def _sa_mlp_body(g_ref, new_ref, w1_ref, b1_ref, w2_ref, b2_ref, w3_ref,
                 b3_ref, out_ref, *, Sblk, Cout, Dpad):
    grows = g_ref[0].reshape(Sblk, NS, Dpad)
    new = new_ref[0]
    sub = jnp.concatenate(
        [new, jnp.zeros((Sblk, Dpad - 3), jnp.float32)], axis=1)
    xrows = (grows - sub[:, None, :]).reshape(Sblk * NS, Dpad)
    h = jnp.maximum(xrows @ w1_ref[...] + b1_ref[...], 0.0)
    h = jnp.maximum(h @ w2_ref[...] + b2_ref[...], 0.0)
    h = jnp.maximum(h @ w3_ref[...] + b3_ref[...], 0.0)
    hm = h.reshape(Sblk, NS, Cout)
    out_ref[0] = jnp.max(hm, axis=1)


def _sa_mlp(g, new_xyz, layers, Sblk):
    B, R, Dpad = g.shape
    S = R // NS
    ws = []
    for li, (Wm, bm) in enumerate(layers):
        cin, cout = Wm.shape
        rows_pad = Dpad if li == 0 else _rup(cin, 8)
        Wp = jnp.zeros((rows_pad, cout), jnp.float32).at[:cin].set(Wm)
        ws.append((Wp, bm.reshape(1, cout)))
    Cout = ws[-1][0].shape[1]
    grid = (B, S // Sblk)
    specs = [
        pl.BlockSpec((1, Sblk * NS, Dpad), lambda b, s: (b, s, 0)),
        pl.BlockSpec((1, Sblk, 3), lambda b, s: (b, s, 0)),
    ]
    args = [g, new_xyz]
    for (Wp, bp) in ws:
        specs.append(pl.BlockSpec(Wp.shape, lambda b, s: (0, 0)))
        specs.append(pl.BlockSpec(bp.shape, lambda b, s: (0, 0)))
        args.extend([Wp, bp])
    return pl.pallas_call(
        functools.partial(_sa_mlp_body, Sblk=Sblk, Cout=Cout, Dpad=Dpad),
        grid=grid,
        compiler_params=pltpu.CompilerParams(
            dimension_semantics=("parallel", "arbitrary")),
        in_specs=specs,
        out_specs=pl.BlockSpec((1, Sblk, Cout), lambda b, s: (b, s, 0)),
        out_shape=jax.ShapeDtypeStruct((B, S, Cout), jnp.float32),
    )(*args)


def _fp_body(x1_ref, q_ref, f2_ref, f1_ref, *rest, S2, C1, Cin_pad, nlayers,
             Nblk, Cout):
    wrefs = rest[:-1]
    out_ref = rest[-1]
    p = x1_ref[0]
    q = q_ref[0]
    dx = p[:, 0:1] - q[0:1, :]
    dy = p[:, 1:2] - q[1:2, :]
    dz = p[:, 2:3] - q[2:3, :]
    d = (dx * dx + dy * dy) + dz * dz
    lane = lax.broadcasted_iota(jnp.int32, (Nblk, S2), 1)

    def extract(dcur):
        dm = jnp.min(dcur, axis=1, keepdims=True)
        im = jnp.min(jnp.where(dcur == dm, lane, S2), axis=1, keepdims=True)
        eq = lane == im
        dnext = jnp.where(eq, 1e30, dcur)
        return dm, eq, dnext

    d1, e1, dc = extract(d)
    d2, e2, dc = extract(dc)
    d3, e3, _ = extract(dc)
    r1 = 1.0 / (d1 + 1e-8)
    r2 = 1.0 / (d2 + 1e-8)
    r3 = 1.0 / (d3 + 1e-8)
    rs = r1 + r2 + r3
    w1 = r1 / rs
    w2 = r2 / rs
    w3 = r3 / rs
    Wm = (jnp.where(e1, w1, 0.0) + jnp.where(e2, w2, 0.0)
          + jnp.where(e3, w3, 0.0))
    interp = jax.lax.dot(Wm, f2_ref[0],
                         precision=jax.lax.Precision.HIGHEST)
    xin = jnp.concatenate([interp, f1_ref[0]], axis=1)
    pad = Cin_pad - xin.shape[1]
    if pad:
        xin = jnp.concatenate(
            [xin, jnp.zeros((Nblk, pad), jnp.float32)], axis=1)
    h = xin
    for i in range(nlayers):
        h = jnp.maximum(h @ wrefs[2 * i][...] + wrefs[2 * i + 1][...], 0.0)
    out_ref[0] = h


def _fp(xyz1, xyz2, feats1, feats2, layers, Nblk):
    B, N1, _ = xyz1.shape
    S2 = xyz2.shape[1]
    C1 = feats1.shape[2]
    C2 = feats2.shape[2]
    q = jnp.transpose(xyz2, (0, 2, 1))
    Cin = C1 + C2
    Cin_pad = _rup(Cin, 8)
    ws = []
    for (Wm, bm) in layers:
        cin, cout = Wm.shape
        Wp = jnp.zeros((_rup(cin, 8), cout), jnp.float32).at[:cin].set(Wm)
        ws.append((Wp, bm.reshape(1, cout)))
    nlayers = len(ws)
    Cout = ws[-1][0].shape[1]
    grid = (B, N1 // Nblk)
    specs = [
        pl.BlockSpec((1, Nblk, 3), lambda b, s: (b, s, 0)),
        pl.BlockSpec((1, 3, S2), lambda b, s: (b, 0, 0)),
        pl.BlockSpec((1, S2, C2), lambda b, s: (b, 0, 0)),
        pl.BlockSpec((1, Nblk, C1), lambda b, s: (b, s, 0)),
    ]
    args = [xyz1, q, feats2, feats1]
    for (Wp, bp) in ws:
        specs.append(pl.BlockSpec(Wp.shape, lambda b, s: (0, 0)))
        specs.append(pl.BlockSpec(bp.shape, lambda b, s: (0, 0)))
        args.extend([Wp, bp])
    return pl.pallas_call(
        functools.partial(_fp_body, S2=S2, C1=C1, Cin_pad=Cin_pad,
                          nlayers=nlayers, Nblk=Nblk, Cout=Cout),
        grid=grid,
        compiler_params=pltpu.CompilerParams(
            dimension_semantics=("parallel", "arbitrary")),
        in_specs=specs,
        out_specs=pl.BlockSpec((1, Nblk, Cout), lambda b, s: (b, s, 0)),
        out_shape=jax.ShapeDtypeStruct((B, N1, Cout), jnp.float32),
    )(*args)


def kernel(pointcloud, params):
    B = pointcloud.shape[0]
    xyz = pointcloud[..., 0:3]
    feats = pointcloud[..., 3:]

    l_xyz = [xyz]
    l_feats = [feats]
    for lvl, (N, S, r, Cf, dims, W, U, Sblk_bq, Sblk_mlp) in enumerate(
            SA_LEVELS):
        cur_xyz = l_xyz[lvl]
        cur_f = l_feats[lvl]
        new_xyz = _fps(cur_xyz, S)
        idx = _ball_query(cur_xyz, new_xyz, r, W, U, Sblk_bq)
        Dpad = _rup(3 + Cf, 16)
        tab = jnp.concatenate([cur_xyz, cur_f], axis=-1)
        tab = jnp.pad(tab, ((0, 0), (0, 0), (0, Dpad - (3 + Cf))))
        tab = tab.reshape(B * N, Dpad)
        glob = (idx + jnp.arange(B, dtype=jnp.int32)[:, None, None] * N)
        rows = _sc_gather_call(tab, glob.reshape(-1))
        g = rows.reshape(B, S * NS, Dpad)
        nf = _sa_mlp(g, new_xyz, params["sa"][lvl], Sblk_mlp)
        l_xyz.append(new_xyz)
        l_feats.append(nf)

    fp_nblk = [1024, 256, 256, 64]
    for i in range(-1, -5, -1):
        l_feats[i - 1] = _fp(
            l_xyz[i - 1], l_xyz[i], l_feats[i - 1], l_feats[i],
            params["fp"][i], fp_nblk[i])
    return l_feats[0]


# speedup vs baseline: 1.1786x; 1.0198x over previous
import functools

import jax
import jax.numpy as jnp
from jax import lax
from jax.experimental import pallas as pl
from jax.experimental.pallas import tpu as pltpu
from jax.experimental.pallas import tpu_sc as plsc

SA_LEVELS = [
    (4096, 1024, 0.1, 3, (6, 32, 32, 64), 32, 8, 512, 512),
    (1024, 256, 0.2, 64, (67, 64, 64, 128), 8, 8, 256, 256),
    (256, 64, 0.4, 128, (131, 128, 128, 256), 2, 8, 64, 64),
    (64, 16, 0.8, 256, (259, 256, 256, 512), 1, 4, 16, 16),
]
NS = 32


def _rup(x, m):
    return (x + m - 1) // m * m


def _fps_body(x_ref, y_ref, z_ref, out_ref, dist_ref, *, S):
    B, N = x_ref.shape
    ids = lax.broadcasted_iota(jnp.int32, (B, N), 1)
    dist_ref[...] = jnp.full((B, N), 1e10, jnp.float32)

    def body(s, cur):
        x = x_ref[...]
        y = y_ref[...]
        z = z_ref[...]
        eq = ids == cur
        cx = jnp.sum(jnp.where(eq, x, 0.0), axis=1, keepdims=True)
        cy = jnp.sum(jnp.where(eq, y, 0.0), axis=1, keepdims=True)
        cz = jnp.sum(jnp.where(eq, z, 0.0), axis=1, keepdims=True)
        cent = jnp.concatenate([cx, cy, cz], axis=1)
        out_ref[pl.ds(s, 1), :, :] = cent[None]
        dx = x - cx
        dy = y - cy
        dz = z - cz
        d = (dx * dx + dy * dy) + dz * dz
        dmin = jnp.minimum(dist_ref[...], d)
        dist_ref[...] = dmin
        m = jnp.max(dmin, axis=1, keepdims=True)
        nxt = jnp.min(jnp.where(dmin == m, ids, N), axis=1, keepdims=True)
        return nxt

    lax.fori_loop(0, S, body, jnp.zeros((B, 1), jnp.int32))


def _fps(xyz, S):
    B, N, _ = xyz.shape
    xt = jnp.transpose(xyz, (2, 0, 1))
    out = pl.pallas_call(
        functools.partial(_fps_body, S=S),
        out_shape=jax.ShapeDtypeStruct((S, B, 3), jnp.float32),
        scratch_shapes=[pltpu.VMEM((B, N), jnp.float32)],
    )(xt[0], xt[1], xt[2])
    return jnp.transpose(out, (1, 0, 2))


def _shift_lanes(a, sh):
    z = jnp.zeros(a.shape[:-1] + (sh,), a.dtype)
    return jnp.concatenate([z, a[..., :-sh]], axis=-1)


def _shift_sub(a, sh):
    z = jnp.zeros((a.shape[0], sh, a.shape[2]), a.dtype)
    return jnp.concatenate([z, a[:, :-sh, :]], axis=1)


def _ballq_body(xt_ref, yt_ref, zt_ref, new_ref, out_ref, *, r2, W, U, Sblk):
    N = xt_ref.shape[2]
    K16 = U * 16
    x = xt_ref[0]
    y = yt_ref[0]
    z = zt_ref[0]
    new = new_ref[0]
    nx = new[:, 0:1]
    ny = new[:, 1:2]
    nz = new[:, 2:3]
    dx = nx - x
    dy = ny - y
    dz = nz - z
    sq = (dx * dx + dy * dy) + dz * dz
    mask = (sq < r2).astype(jnp.float32)

    m2 = mask.reshape(Sblk * W, K16)
    li = lax.broadcasted_iota(jnp.int32, (K16, 2 * U), 0)
    ui = lax.broadcasted_iota(jnp.int32, (K16, 2 * U), 1)
    ind = (li // 16) == (ui % U)
    kcnt = jnp.where(ind & (ui < U), 1.0, 0.0)
    kbit = jnp.where(ind & (ui >= U), (1 << (li % 16)).astype(jnp.float32), 0.0)
    tb = m2 @ (kcnt + kbit)
    tb3 = tb.reshape(Sblk, W, 2 * U)
    t2T = jnp.swapaxes(tb3[:, :, :U], 1, 2)
    bitsT = jnp.swapaxes(tb3[:, :, U:], 1, 2).astype(jnp.int32)

    t1 = jnp.sum(t2T, axis=1)
    H1 = t1
    sh = 1
    while sh < W:
        H1 = H1 + _shift_lanes(H1, sh)
        sh *= 2
    H1x = (H1 - t1).astype(jnp.int32)
    H1i = H1.astype(jnp.int32)
    count = H1i[:, W - 1:W]

    karr = lax.broadcasted_iota(jnp.int32, (Sblk, NS), 1)
    if W > 1:
        ws = jnp.sum((H1i[:, :, None] <= karr[:, None, :]).astype(jnp.int32),
                     axis=1)
        wsc = jnp.minimum(ws, W - 1)
        base1 = jnp.take_along_axis(H1x, wsc, axis=1)
        idxw = jnp.broadcast_to(wsc[:, None, :], (Sblk, U, NS))
        t2sel = jnp.take_along_axis(t2T.astype(jnp.int32), idxw, axis=2)
        bsel0 = jnp.take_along_axis(bitsT, idxw, axis=2)
    else:
        wsc = jnp.zeros((Sblk, NS), jnp.int32)
        base1 = jnp.zeros((Sblk, NS), jnp.int32)
        t2sel = jnp.broadcast_to(t2T.astype(jnp.int32), (Sblk, U, NS))
        bsel0 = jnp.broadcast_to(bitsT, (Sblk, U, NS))
    r1 = karr - base1

    if U > 1:
        H2 = t2sel
        sh = 1
        while sh < U:
            H2 = H2 + _shift_sub(H2, sh)
            sh *= 2
        H2x = H2 - t2sel
        us = jnp.sum((H2 <= r1[:, None, :]).astype(jnp.int32), axis=1)
        usc = jnp.minimum(us, U - 1)
        base2 = jnp.take_along_axis(H2x, usc[:, None, :], axis=1)[:, 0, :]
        bsel = jnp.take_along_axis(bsel0, usc[:, None, :], axis=1)[:, 0, :]
    else:
        usc = jnp.zeros((Sblk, NS), jnp.int32)
        base2 = jnp.zeros((Sblk, NS), jnp.int32)
        bsel = bsel0[:, 0, :]
    r2i = r1 - base2

    pos = jnp.zeros((Sblk, NS), jnp.int32)
    rem = r2i
    b = bsel
    for width in (8, 4, 2, 1):
        lowc = lax.population_count(b & ((1 << width) - 1))
        take = (rem >= lowc).astype(jnp.int32)
        pos = pos + take * width
        rem = rem - take * lowc
        b = lax.shift_right_logical(b, take * width)

    outk = wsc * (U * 16) + usc * 16 + pos
    outk = jnp.where(karr < count, outk, outk[:, 0:1])
    out_ref[0] = outk


def _ball_query(xyz, new_xyz, r, W, U, Sblk):
    B, N, _ = xyz.shape
    S = new_xyz.shape[1]
    xt = jnp.transpose(xyz, (2, 0, 1))[:, :, None, :]
    grid = (B, S // Sblk)
    return pl.pallas_call(
        functools.partial(_ballq_body, r2=r * r, W=W, U=U, Sblk=Sblk),
        grid=grid,
        compiler_params=pltpu.CompilerParams(
            dimension_semantics=("parallel", "arbitrary")),
        in_specs=[
            pl.BlockSpec((1, 1, N), lambda b, s: (b, 0, 0)),
            pl.BlockSpec((1, 1, N), lambda b, s: (b, 0, 0)),
            pl.BlockSpec((1, 1, N), lambda b, s: (b, 0, 0)),
            pl.BlockSpec((1, Sblk, 3), lambda b, s: (b, s, 0)),
        ],
        out_specs=pl.BlockSpec((1, Sblk, NS), lambda b, s: (b, s, 0)),
        out_shape=jax.ShapeDtypeStruct((B, S, NS), jnp.int32),
    )(xt[0], xt[1], xt[2], new_xyz)


def _sc_gather_call(table, idx_flat):
    V, D = table.shape
    Btot = idx_flat.shape[0]
    info = plsc.get_sparse_core_info()
    NW = info.num_cores * info.num_subcores
    b_per_w = Btot // NW
    CH = b_per_w
    while CH * D * 4 > 200_000:
        CH //= 2
    nch = b_per_w // CH
    mesh = plsc.VectorSubcoreMesh(core_axis_name="c", subcore_axis_name="s")

    @functools.partial(
        pl.kernel, mesh=mesh,
        compiler_params=pltpu.CompilerParams(use_tc_tiling_on_sc=False),
        out_type=jax.ShapeDtypeStruct((Btot, D), jnp.float32),
        scratch_types=[
            pltpu.VMEM((CH,), jnp.int32),
            pltpu.VMEM((CH,), jnp.int32),
            pltpu.VMEM((CH, D), jnp.float32),
            pltpu.VMEM((CH, D), jnp.float32),
            pltpu.SemaphoreType.DMA,
            pltpu.SemaphoreType.DMA,
        ],
    )
    def k(table_hbm, idx_hbm, out_hbm, idx_v0, idx_v1, rows_v0, rows_v1,
          sem0, sem1):
        wid = lax.axis_index("s") * info.num_cores + lax.axis_index("c")
        base = wid * b_per_w
        idx_bufs = (idx_v0, idx_v1)
        row_bufs = (rows_v0, rows_v1)
        sems = (sem0, sem1)
        copies = [None, None]
        pltpu.sync_copy(idx_hbm.at[pl.ds(base, CH)], idx_v0)
        copies[0] = pltpu.async_copy(table_hbm.at[idx_v0], rows_v0, sem0)
        for c in range(nch):
            nxt = (c + 1) % 2
            if c + 1 < nch:
                off = base + (c + 1) * CH
                pltpu.sync_copy(idx_hbm.at[pl.ds(off, CH)], idx_bufs[nxt])
                copies[nxt] = pltpu.async_copy(
                    table_hbm.at[idx_bufs[nxt]], row_bufs[nxt], sems[nxt])
            copies[c % 2].wait()
            pltpu.sync_copy(row_bufs[c % 2],
                            out_hbm.at[pl.ds(base + c * CH, CH)])

    return k(table, idx_flat)


def _sa_mlp_body(g_ref, new_ref, w1_ref, b1_ref, w2_ref, b2_ref, w3_ref,
                 b3_ref, out_ref, *, Sblk, Cout, Dpad):
    grows = g_ref[0].reshape(Sblk, NS, Dpad)
    new = new_ref[0]
    sub = jnp.concatenate(
        [new, jnp.zeros((Sblk, Dpad - 3), jnp.float32)], axis=1)
    xrows = (grows - sub[:, None, :]).reshape(Sblk * NS, Dpad)
    h = jnp.maximum(xrows @ w1_ref[...] + b1_ref[...], 0.0)
    h = jnp.maximum(h @ w2_ref[...] + b2_ref[...], 0.0)
    h = jnp.maximum(h @ w3_ref[...] + b3_ref[...], 0.0)
    hm = h.reshape(Sblk, NS, Cout)
    out_ref[0] = jnp.max(hm, axis=1)


def _sa_mlp(g, new_xyz, layers, Sblk):
    B, R, Dpad = g.shape
    S = R // NS
    ws = []
    for li, (Wm, bm) in enumerate(layers):
        cin, cout = Wm.shape
        rows_pad = Dpad if li == 0 else _rup(cin, 8)
        Wp = jnp.zeros((rows_pad, cout), jnp.float32).at[:cin].set(Wm)
        ws.append((Wp, bm.reshape(1, cout)))
    Cout = ws[-1][0].shape[1]
    grid = (B, S // Sblk)
    specs = [
        pl.BlockSpec((1, Sblk * NS, Dpad), lambda b, s: (b, s, 0)),
        pl.BlockSpec((1, Sblk, 3), lambda b, s: (b, s, 0)),
    ]
    args = [g, new_xyz]
    for (Wp, bp) in ws:
        specs.append(pl.BlockSpec(Wp.shape, lambda b, s: (0, 0)))
        specs.append(pl.BlockSpec(bp.shape, lambda b, s: (0, 0)))
        args.extend([Wp, bp])
    return pl.pallas_call(
        functools.partial(_sa_mlp_body, Sblk=Sblk, Cout=Cout, Dpad=Dpad),
        grid=grid,
        compiler_params=pltpu.CompilerParams(
            dimension_semantics=("parallel", "arbitrary")),
        in_specs=specs,
        out_specs=pl.BlockSpec((1, Sblk, Cout), lambda b, s: (b, s, 0)),
        out_shape=jax.ShapeDtypeStruct((B, S, Cout), jnp.float32),
    )(*args)


def _fp_body(x1_ref, q_ref, f2_ref, f1_ref, *rest, S2, C1, Cin_pad, nlayers,
             Nblk, Cout):
    wrefs = rest[:-1]
    out_ref = rest[-1]
    p = x1_ref[0]
    q = q_ref[0]
    dx = p[:, 0:1] - q[0:1, :]
    dy = p[:, 1:2] - q[1:2, :]
    dz = p[:, 2:3] - q[2:3, :]
    d = (dx * dx + dy * dy) + dz * dz
    lane = lax.broadcasted_iota(jnp.int32, (Nblk, S2), 1)

    def extract(dcur):
        dm = jnp.min(dcur, axis=1, keepdims=True)
        im = jnp.min(jnp.where(dcur == dm, lane, S2), axis=1, keepdims=True)
        eq = lane == im
        dnext = jnp.where(eq, 1e30, dcur)
        return dm, eq, dnext

    d1, e1, dc = extract(d)
    d2, e2, dc = extract(dc)
    d3, e3, _ = extract(dc)
    r1 = 1.0 / (d1 + 1e-8)
    r2 = 1.0 / (d2 + 1e-8)
    r3 = 1.0 / (d3 + 1e-8)
    rs = r1 + r2 + r3
    w1 = r1 / rs
    w2 = r2 / rs
    w3 = r3 / rs
    Wm = (jnp.where(e1, w1, 0.0) + jnp.where(e2, w2, 0.0)
          + jnp.where(e3, w3, 0.0))
    interp = jax.lax.dot(Wm, f2_ref[0],
                         precision=jax.lax.Precision.HIGHEST)
    xin = jnp.concatenate([interp, f1_ref[0]], axis=1)
    pad = Cin_pad - xin.shape[1]
    if pad:
        xin = jnp.concatenate(
            [xin, jnp.zeros((Nblk, pad), jnp.float32)], axis=1)
    h = xin
    for i in range(nlayers):
        h = jnp.maximum(h @ wrefs[2 * i][...] + wrefs[2 * i + 1][...], 0.0)
    out_ref[0] = h


def _fp(xyz1, xyz2, feats1, feats2, layers, Nblk):
    B, N1, _ = xyz1.shape
    S2 = xyz2.shape[1]
    C1 = feats1.shape[2]
    C2 = feats2.shape[2]
    q = jnp.transpose(xyz2, (0, 2, 1))
    Cin = C1 + C2
    Cin_pad = _rup(Cin, 8)
    ws = []
    for (Wm, bm) in layers:
        cin, cout = Wm.shape
        Wp = jnp.zeros((_rup(cin, 8), cout), jnp.float32).at[:cin].set(Wm)
        ws.append((Wp, bm.reshape(1, cout)))
    nlayers = len(ws)
    Cout = ws[-1][0].shape[1]
    grid = (B, N1 // Nblk)
    specs = [
        pl.BlockSpec((1, Nblk, 3), lambda b, s: (b, s, 0)),
        pl.BlockSpec((1, 3, S2), lambda b, s: (b, 0, 0)),
        pl.BlockSpec((1, S2, C2), lambda b, s: (b, 0, 0)),
        pl.BlockSpec((1, Nblk, C1), lambda b, s: (b, s, 0)),
    ]
    args = [xyz1, q, feats2, feats1]
    for (Wp, bp) in ws:
        specs.append(pl.BlockSpec(Wp.shape, lambda b, s: (0, 0)))
        specs.append(pl.BlockSpec(bp.shape, lambda b, s: (0, 0)))
        args.extend([Wp, bp])
    return pl.pallas_call(
        functools.partial(_fp_body, S2=S2, C1=C1, Cin_pad=Cin_pad,
                          nlayers=nlayers, Nblk=Nblk, Cout=Cout),
        grid=grid,
        compiler_params=pltpu.CompilerParams(
            dimension_semantics=("parallel", "arbitrary")),
        in_specs=specs,
        out_specs=pl.BlockSpec((1, Nblk, Cout), lambda b, s: (b, s, 0)),
        out_shape=jax.ShapeDtypeStruct((B, N1, Cout), jnp.float32),
    )(*args)


def kernel(pointcloud, params):
    B = pointcloud.shape[0]
    xyz = pointcloud[..., 0:3]
    feats = pointcloud[..., 3:]

    l_xyz = [xyz]
    l_feats = [feats]
    for lvl, (N, S, r, Cf, dims, W, U, Sblk_bq, Sblk_mlp) in enumerate(
            SA_LEVELS):
        cur_xyz = l_xyz[lvl]
        cur_f = l_feats[lvl]
        new_xyz = _fps(cur_xyz, S)
        idx = _ball_query(cur_xyz, new_xyz, r, W, U, Sblk_bq)
        Dpad = _rup(3 + Cf, 16)
        tab = jnp.concatenate([cur_xyz, cur_f], axis=-1)
        tab = jnp.pad(tab, ((0, 0), (0, 0), (0, Dpad - (3 + Cf))))
        tab = tab.reshape(B * N, Dpad)
        glob = (idx + jnp.arange(B, dtype=jnp.int32)[:, None, None] * N)
        rows = _sc_gather_call(tab, glob.reshape(-1))
        g = rows.reshape(B, S * NS, Dpad)
        nf = _sa_mlp(g, new_xyz, params["sa"][lvl], Sblk_mlp)
        l_xyz.append(new_xyz)
        l_feats.append(nf)

    fp_nblk = [2048, 512, 256, 64]
    for i in range(-1, -5, -1):
        l_feats[i - 1] = _fp(
            l_xyz[i - 1], l_xyz[i], l_feats[i - 1], l_feats[i],
            params["fp"][i], fp_nblk[i])
    return l_feats[0]
